# baseline (speedup 1.0000x reference)
import jax
import jax.numpy as jnp
from jax import lax
from jax.experimental import pallas as pl
from jax.experimental.pallas import tpu as pltpu
from jax.experimental.pallas import tpu_sc as plsc

B = 16384
H = 64
OUT = 128
SUB_V = 6
OBJ_V = 1000
NC = 2
NS = 16
NW = NC * NS
B_PER = B // NW
QUART = B_PER // 4
GRP = 16
STRIDE = H + 1
BK = 4096


def _sc_gather_body(sid, tid, rid, tcat, out,
                    tcat_v, sid_v, tid_v, rid_v,
                    st0, to0, ro0, st1, to1, ro1, sem, wsem):
    wid = lax.axis_index("s") * NC + lax.axis_index("c")
    base = wid * B_PER
    stage = [pltpu.async_copy(tcat, tcat_v, sem),
             pltpu.async_copy(sid.at[pl.ds(base, B_PER)], sid_v, sem),
             pltpu.async_copy(tid.at[pl.ds(base, B_PER)], tid_v, sem),
             pltpu.async_copy(rid.at[pl.ds(base, B_PER)], rid_v, sem)]
    for c in stage:
        c.wait()
    bufsets = ((st0, to0, ro0), (st1, to1, ro1))
    wcopies = []
    for q in range(B_PER // QUART):
        bufs = bufsets[q % 2]
        if q >= 2:
            for _ in range(3):
                wcopies.pop(0).wait()

        @plsc.parallel_loop(0, QUART // GRP, unroll=4)
        def _gather_groups(g, _q=q, _bufs=bufs):
            off = _q * QUART + g * GRP
            for ids_ref, row_off, buf_ref in ((sid_v, 0, _bufs[0]),
                                              (tid_v, SUB_V, _bufs[1]),
                                              (rid_v, SUB_V, _bufs[2])):
                fidx = (ids_ref[pl.ds(off, GRP)] + row_off) * STRIDE
                for c0 in range(0, H, 16):
                    xs = [plsc.load_gather(tcat_v, [fidx + c])
                          for c in range(c0, c0 + 16)]
                    for k in range(0, 16, 2):
                        xp = plsc.pack(xs[k], xs[k + 1],
                                       format=plsc.PackFormat.INTERLEAVED)
                        buf_ref[(c0 + k) // 2,
                                pl.ds(g * 2 * GRP, 2 * GRP)] = xp
        dst = pl.ds(2 * (base + q * QUART), 2 * QUART)
        for t in range(3):
            wcopies.append(pltpu.async_copy(bufs[t], out.at[t, :, dst], wsem))
    for c in wcopies:
        c.wait()


def _dot_t(a, w):
    return lax.dot_general(a, w, (((0,), (0,)), ((), ())),
                           preferred_element_type=jnp.float32)


def _dot(a, w):
    return jnp.dot(a, w, preferred_element_type=jnp.float32)


def _tail_body(g3, tpos, rpos, w_sub, b_sub, w_obj, b_obj,
               w_pos1, b_pos1, w_pos2, b_pos2, w_c1, b_c1,
               w_c2, b_c2, out):
    wc1_s = w_c1[0:64, :]
    wc1_t = w_c1[64:128, :]
    wc1_p1 = w_c1[128:192, :]
    wc1_r = w_c1[192:256, :]
    wc1_p2 = w_c1[256:320, :]
    a_s = _dot(w_sub[...], wc1_s)
    a_t = _dot(w_obj[...], wc1_t)
    a_r = _dot(w_obj[...], wc1_r)
    m_t = _dot(w_pos2[...], wc1_p1)
    m_r = _dot(w_pos2[...], wc1_p2)
    cvec = (b_c1[...]
            + _dot(b_sub[...], wc1_s)
            + _dot(b_obj[...], wc1_t) + _dot(b_obj[...], wc1_r)
            + _dot(b_pos2[...], wc1_p1) + _dot(b_pos2[...], wc1_p2))
    at_ = jnp.maximum(_dot(tpos[...], w_pos1[...]) + b_pos1[...], 0.0)
    ar_ = jnp.maximum(_dot(rpos[...], w_pos1[...]) + b_pos1[...], 0.0)
    def branch(y_bf, a):
        y = jnp.maximum(y_bf, 0).astype(jnp.float32)
        a2 = a.reshape(H // 2, 2, OUT)
        z_e = _dot_t(y, a2[:, 0, :]).reshape(BK, 2, OUT)[:, 0, :]
        z_o = _dot_t(y, a2[:, 1, :]).reshape(BK, 2, OUT)[:, 1, :]
        return z_e + z_o

    h1 = (branch(g3[0], a_s)
          + branch(g3[1], a_t)
          + branch(g3[2], a_r)
          + _dot(at_, m_t)
          + _dot(ar_, m_r)
          + cvec)
    h = jnp.maximum(h1, 0.0)
    out[...] = jnp.maximum(_dot(h, w_c2[...]) + b_c2[...], 0.0)


def kernel(subtask_type_id, target_obj_type_id, target_obj_pos,
           receptacle_obj_type_id, receptacle_obj_pos,
           emb_sub, W_sub, b_sub, emb_obj, W_obj, b_obj,
           W_pos1, b_pos1, W_pos2, b_pos2, W_c1, b_c1, W_c2, b_c2):
    f32 = jnp.float32

    ecat = jnp.pad(jnp.concatenate([emb_sub, emb_obj], axis=0),
                   ((0, 0), (0, 1))).reshape(-1)

    mesh = plsc.VectorSubcoreMesh(core_axis_name="c", subcore_axis_name="s")
    g3 = pl.kernel(
        _sc_gather_body,
        out_type=jax.ShapeDtypeStruct((3, H // 2, 2 * B), jnp.bfloat16),
        mesh=mesh,
        compiler_params=pltpu.CompilerParams(needs_layout_passes=False),
        scratch_types=[
            pltpu.VMEM(((SUB_V + OBJ_V) * STRIDE,), f32),
            pltpu.VMEM((B_PER,), jnp.int32),
            pltpu.VMEM((B_PER,), jnp.int32),
            pltpu.VMEM((B_PER,), jnp.int32),
            pltpu.VMEM((H // 2, 2 * QUART), jnp.bfloat16),
            pltpu.VMEM((H // 2, 2 * QUART), jnp.bfloat16),
            pltpu.VMEM((H // 2, 2 * QUART), jnp.bfloat16),
            pltpu.VMEM((H // 2, 2 * QUART), jnp.bfloat16),
            pltpu.VMEM((H // 2, 2 * QUART), jnp.bfloat16),
            pltpu.VMEM((H // 2, 2 * QUART), jnp.bfloat16),
            pltpu.SemaphoreType.DMA,
            pltpu.SemaphoreType.DMA,
        ],
    )(subtask_type_id, target_obj_type_id, receptacle_obj_type_id, ecat)

    grid = (B // BK,)
    out = pl.pallas_call(
        _tail_body,
        grid=grid,
        in_specs=[
            pl.BlockSpec((3, H // 2, 2 * BK), lambda i: (0, 0, i)),
            pl.BlockSpec((BK, 3), lambda i: (i, 0)),
            pl.BlockSpec((BK, 3), lambda i: (i, 0)),
            pl.BlockSpec((H, H), lambda i: (0, 0)),
            pl.BlockSpec((1, H), lambda i: (0, 0)),
            pl.BlockSpec((H, H), lambda i: (0, 0)),
            pl.BlockSpec((1, H), lambda i: (0, 0)),
            pl.BlockSpec((3, H), lambda i: (0, 0)),
            pl.BlockSpec((1, H), lambda i: (0, 0)),
            pl.BlockSpec((H, H), lambda i: (0, 0)),
            pl.BlockSpec((1, H), lambda i: (0, 0)),
            pl.BlockSpec((320, OUT), lambda i: (0, 0)),
            pl.BlockSpec((1, OUT), lambda i: (0, 0)),
            pl.BlockSpec((OUT, OUT), lambda i: (0, 0)),
            pl.BlockSpec((1, OUT), lambda i: (0, 0)),
        ],
        out_specs=pl.BlockSpec((BK, OUT), lambda i: (i, 0)),
        out_shape=jax.ShapeDtypeStruct((B, OUT), f32),
    )(g3, target_obj_pos, receptacle_obj_pos,
      W_sub, b_sub.reshape(1, H), W_obj, b_obj.reshape(1, H),
      W_pos1, b_pos1.reshape(1, H), W_pos2, b_pos2.reshape(1, H),
      W_c1, b_c1.reshape(1, OUT), W_c2, b_c2.reshape(1, OUT))
    return out


# speedup vs baseline: 1.6479x; 1.6479x over previous
import jax
import jax.numpy as jnp
from jax import lax
from jax.experimental import pallas as pl
from jax.experimental.pallas import tpu as pltpu
from jax.experimental.pallas import tpu_sc as plsc

B = 16384
H = 64
OUT = 128
SUB_V = 6
OBJ_V = 1000
NC = 2
NS = 16
NW = NC * NS
B_PER = B // NW
QUART = B_PER // 4
GRP = 16
STRIDE = H + 1
BK = 8192


def _sc_gather_body(sid, tid, rid, tcat, out,
                    tcat_v, sid_v, tid_v, rid_v,
                    st0, to0, ro0, st1, to1, ro1, sem, wsem):
    wid = lax.axis_index("s") * NC + lax.axis_index("c")
    base = wid * B_PER
    stage = [pltpu.async_copy(tcat, tcat_v, sem),
             pltpu.async_copy(sid.at[pl.ds(base, B_PER)], sid_v, sem),
             pltpu.async_copy(tid.at[pl.ds(base, B_PER)], tid_v, sem),
             pltpu.async_copy(rid.at[pl.ds(base, B_PER)], rid_v, sem)]
    for c in stage:
        c.wait()
    bufsets = ((st0, to0, ro0), (st1, to1, ro1))
    wcopies = []
    for q in range(B_PER // QUART):
        bufs = bufsets[q % 2]
        if q >= 2:
            for _ in range(3):
                wcopies.pop(0).wait()

        @plsc.parallel_loop(0, QUART // GRP, unroll=4)
        def _gather_groups(g, _q=q, _bufs=bufs):
            off = _q * QUART + g * GRP
            for ids_ref, row_off, buf_ref in ((sid_v, 0, _bufs[0]),
                                              (tid_v, SUB_V, _bufs[1]),
                                              (rid_v, SUB_V, _bufs[2])):
                fidx = (ids_ref[pl.ds(off, GRP)] + row_off) * STRIDE
                for c0 in range(0, H, 16):
                    xs = [plsc.load_gather(tcat_v, [fidx + c])
                          for c in range(c0, c0 + 16)]
                    for k, x in enumerate(xs):
                        buf_ref[c0 + k, pl.ds(g * GRP, GRP)] = x
        dst = pl.ds(base + q * QUART, QUART)
        for t in range(3):
            wcopies.append(pltpu.async_copy(bufs[t], out.at[t, :, dst], wsem))
    for c in wcopies:
        c.wait()


def _dot_t(a, w):
    return lax.dot_general(a, w, (((0,), (0,)), ((), ())),
                           preferred_element_type=jnp.float32)


def _dot(a, w):
    return jnp.dot(a, w, preferred_element_type=jnp.float32)


def _tail_body(g3, tpos, rpos, w_sub, b_sub, w_obj, b_obj,
               w_pos1, b_pos1, w_pos2, b_pos2, w_c1, b_c1,
               w_c2, b_c2, out):
    wc1_s = w_c1[0:64, :]
    wc1_t = w_c1[64:128, :]
    wc1_p1 = w_c1[128:192, :]
    wc1_r = w_c1[192:256, :]
    wc1_p2 = w_c1[256:320, :]
    a_s = _dot(w_sub[...], wc1_s)
    a_t = _dot(w_obj[...], wc1_t)
    a_r = _dot(w_obj[...], wc1_r)
    m_t = _dot(w_pos2[...], wc1_p1)
    m_r = _dot(w_pos2[...], wc1_p2)
    cvec = (b_c1[...]
            + _dot(b_sub[...], wc1_s)
            + _dot(b_obj[...], wc1_t) + _dot(b_obj[...], wc1_r)
            + _dot(b_pos2[...], wc1_p1) + _dot(b_pos2[...], wc1_p2))
    at_ = jnp.maximum(_dot(tpos[...], w_pos1[...]) + b_pos1[...], 0.0)
    ar_ = jnp.maximum(_dot(rpos[...], w_pos1[...]) + b_pos1[...], 0.0)
    h1 = (_dot_t(jnp.maximum(g3[0], 0.0), a_s)
          + _dot_t(jnp.maximum(g3[1], 0.0), a_t)
          + _dot_t(jnp.maximum(g3[2], 0.0), a_r)
          + _dot(at_, m_t)
          + _dot(ar_, m_r)
          + cvec)
    h = jnp.maximum(h1, 0.0)
    out[...] = jnp.maximum(_dot(h, w_c2[...]) + b_c2[...], 0.0)


def kernel(subtask_type_id, target_obj_type_id, target_obj_pos,
           receptacle_obj_type_id, receptacle_obj_pos,
           emb_sub, W_sub, b_sub, emb_obj, W_obj, b_obj,
           W_pos1, b_pos1, W_pos2, b_pos2, W_c1, b_c1, W_c2, b_c2):
    f32 = jnp.float32

    ecat = jnp.pad(jnp.concatenate([emb_sub, emb_obj], axis=0),
                   ((0, 0), (0, 1))).reshape(-1)

    mesh = plsc.VectorSubcoreMesh(core_axis_name="c", subcore_axis_name="s")
    g3 = pl.kernel(
        _sc_gather_body,
        out_type=jax.ShapeDtypeStruct((3, H, B), f32),
        mesh=mesh,
        compiler_params=pltpu.CompilerParams(needs_layout_passes=False),
        scratch_types=[
            pltpu.VMEM(((SUB_V + OBJ_V) * STRIDE,), f32),
            pltpu.VMEM((B_PER,), jnp.int32),
            pltpu.VMEM((B_PER,), jnp.int32),
            pltpu.VMEM((B_PER,), jnp.int32),
            pltpu.VMEM((H, QUART), f32),
            pltpu.VMEM((H, QUART), f32),
            pltpu.VMEM((H, QUART), f32),
            pltpu.VMEM((H, QUART), f32),
            pltpu.VMEM((H, QUART), f32),
            pltpu.VMEM((H, QUART), f32),
            pltpu.SemaphoreType.DMA,
            pltpu.SemaphoreType.DMA,
        ],
    )(subtask_type_id, target_obj_type_id, receptacle_obj_type_id, ecat)

    grid = (B // BK,)
    out = pl.pallas_call(
        _tail_body,
        grid=grid,
        in_specs=[
            pl.BlockSpec((3, H, BK), lambda i: (0, 0, i)),
            pl.BlockSpec((BK, 3), lambda i: (i, 0)),
            pl.BlockSpec((BK, 3), lambda i: (i, 0)),
            pl.BlockSpec((H, H), lambda i: (0, 0)),
            pl.BlockSpec((1, H), lambda i: (0, 0)),
            pl.BlockSpec((H, H), lambda i: (0, 0)),
            pl.BlockSpec((1, H), lambda i: (0, 0)),
            pl.BlockSpec((3, H), lambda i: (0, 0)),
            pl.BlockSpec((1, H), lambda i: (0, 0)),
            pl.BlockSpec((H, H), lambda i: (0, 0)),
            pl.BlockSpec((1, H), lambda i: (0, 0)),
            pl.BlockSpec((320, OUT), lambda i: (0, 0)),
            pl.BlockSpec((1, OUT), lambda i: (0, 0)),
            pl.BlockSpec((OUT, OUT), lambda i: (0, 0)),
            pl.BlockSpec((1, OUT), lambda i: (0, 0)),
        ],
        out_specs=pl.BlockSpec((BK, OUT), lambda i: (i, 0)),
        out_shape=jax.ShapeDtypeStruct((B, OUT), f32),
    )(g3, target_obj_pos, receptacle_obj_pos,
      W_sub, b_sub.reshape(1, H), W_obj, b_obj.reshape(1, H),
      W_pos1, b_pos1.reshape(1, H), W_pos2, b_pos2.reshape(1, H),
      W_c1, b_c1.reshape(1, OUT), W_c2, b_c2.reshape(1, OUT))
    return out


# speedup vs baseline: 1.6710x; 1.0140x over previous
import jax
import jax.numpy as jnp
from jax import lax
from jax.experimental import pallas as pl
from jax.experimental.pallas import tpu as pltpu
from jax.experimental.pallas import tpu_sc as plsc

B = 16384
H = 64
OUT = 128
SUB_V = 6
OBJ_V = 1000
NC = 2
NS = 16
NW = NC * NS
B_PER = B // NW
QUART = B_PER // 4
GRP = 16
STRIDE = H + 1
BK = 4096


def _sc_gather_body(sid, tid, rid, tcat, out,
                    tcat_v, sid_v, tid_v, rid_v,
                    st0, to0, ro0, st1, to1, ro1, sem, wsem):
    wid = lax.axis_index("s") * NC + lax.axis_index("c")
    base = wid * B_PER
    stage = [pltpu.async_copy(tcat, tcat_v, sem),
             pltpu.async_copy(sid.at[pl.ds(base, B_PER)], sid_v, sem),
             pltpu.async_copy(tid.at[pl.ds(base, B_PER)], tid_v, sem),
             pltpu.async_copy(rid.at[pl.ds(base, B_PER)], rid_v, sem)]
    for c in stage:
        c.wait()
    bufsets = ((st0, to0, ro0), (st1, to1, ro1))
    wcopies = []
    for q in range(B_PER // QUART):
        bufs = bufsets[q % 2]
        if q >= 2:
            for _ in range(3):
                wcopies.pop(0).wait()

        @plsc.parallel_loop(0, QUART // GRP, unroll=4)
        def _gather_groups(g, _q=q, _bufs=bufs):
            off = _q * QUART + g * GRP
            for ids_ref, row_off, buf_ref in ((sid_v, 0, _bufs[0]),
                                              (tid_v, SUB_V, _bufs[1]),
                                              (rid_v, SUB_V, _bufs[2])):
                fidx = (ids_ref[pl.ds(off, GRP)] + row_off) * STRIDE
                for c0 in range(0, H, 16):
                    xs = [plsc.load_gather(tcat_v, [fidx + c])
                          for c in range(c0, c0 + 16)]
                    for k, x in enumerate(xs):
                        buf_ref[c0 + k, pl.ds(g * GRP, GRP)] = x
        dst = pl.ds(base + q * QUART, QUART)
        for t in range(3):
            wcopies.append(pltpu.async_copy(bufs[t], out.at[t, :, dst], wsem))
    for c in wcopies:
        c.wait()


def _dot_t(a, w):
    return lax.dot_general(a, w, (((0,), (0,)), ((), ())),
                           preferred_element_type=jnp.float32)


def _dot(a, w):
    return jnp.dot(a, w, preferred_element_type=jnp.float32)


def _tail_body(g3, tpos, rpos, w_sub, b_sub, w_obj, b_obj,
               w_pos1, b_pos1, w_pos2, b_pos2, w_c1, b_c1,
               w_c2, b_c2, out):
    wc1_s = w_c1[0:64, :]
    wc1_t = w_c1[64:128, :]
    wc1_p1 = w_c1[128:192, :]
    wc1_r = w_c1[192:256, :]
    wc1_p2 = w_c1[256:320, :]
    a_s = _dot(w_sub[...], wc1_s)
    a_t = _dot(w_obj[...], wc1_t)
    a_r = _dot(w_obj[...], wc1_r)
    m_t = _dot(w_pos2[...], wc1_p1)
    m_r = _dot(w_pos2[...], wc1_p2)
    cvec = (b_c1[...]
            + _dot(b_sub[...], wc1_s)
            + _dot(b_obj[...], wc1_t) + _dot(b_obj[...], wc1_r)
            + _dot(b_pos2[...], wc1_p1) + _dot(b_pos2[...], wc1_p2))
    at_ = jnp.maximum(_dot(tpos[...], w_pos1[...]) + b_pos1[...], 0.0)
    ar_ = jnp.maximum(_dot(rpos[...], w_pos1[...]) + b_pos1[...], 0.0)
    bf16 = jnp.bfloat16
    h1 = (_dot_t(jnp.maximum(g3[0], 0.0).astype(bf16), a_s.astype(bf16))
          + _dot_t(jnp.maximum(g3[1], 0.0).astype(bf16), a_t.astype(bf16))
          + _dot_t(jnp.maximum(g3[2], 0.0).astype(bf16), a_r.astype(bf16))
          + _dot(at_, m_t)
          + _dot(ar_, m_r)
          + cvec)
    h = jnp.maximum(h1, 0.0)
    out[...] = jnp.maximum(
        _dot(h.astype(bf16), w_c2[...].astype(bf16)) + b_c2[...], 0.0)


def kernel(subtask_type_id, target_obj_type_id, target_obj_pos,
           receptacle_obj_type_id, receptacle_obj_pos,
           emb_sub, W_sub, b_sub, emb_obj, W_obj, b_obj,
           W_pos1, b_pos1, W_pos2, b_pos2, W_c1, b_c1, W_c2, b_c2):
    f32 = jnp.float32

    ecat = jnp.pad(jnp.concatenate([emb_sub, emb_obj], axis=0),
                   ((0, 0), (0, 1))).reshape(-1)

    mesh = plsc.VectorSubcoreMesh(core_axis_name="c", subcore_axis_name="s")
    g3 = pl.kernel(
        _sc_gather_body,
        out_type=jax.ShapeDtypeStruct((3, H, B), f32),
        mesh=mesh,
        compiler_params=pltpu.CompilerParams(needs_layout_passes=False),
        scratch_types=[
            pltpu.VMEM(((SUB_V + OBJ_V) * STRIDE,), f32),
            pltpu.VMEM((B_PER,), jnp.int32),
            pltpu.VMEM((B_PER,), jnp.int32),
            pltpu.VMEM((B_PER,), jnp.int32),
            pltpu.VMEM((H, QUART), f32),
            pltpu.VMEM((H, QUART), f32),
            pltpu.VMEM((H, QUART), f32),
            pltpu.VMEM((H, QUART), f32),
            pltpu.VMEM((H, QUART), f32),
            pltpu.VMEM((H, QUART), f32),
            pltpu.SemaphoreType.DMA,
            pltpu.SemaphoreType.DMA,
        ],
    )(subtask_type_id, target_obj_type_id, receptacle_obj_type_id, ecat)

    grid = (B // BK,)
    out = pl.pallas_call(
        _tail_body,
        grid=grid,
        in_specs=[
            pl.BlockSpec((3, H, BK), lambda i: (0, 0, i)),
            pl.BlockSpec((BK, 3), lambda i: (i, 0)),
            pl.BlockSpec((BK, 3), lambda i: (i, 0)),
            pl.BlockSpec((H, H), lambda i: (0, 0)),
            pl.BlockSpec((1, H), lambda i: (0, 0)),
            pl.BlockSpec((H, H), lambda i: (0, 0)),
            pl.BlockSpec((1, H), lambda i: (0, 0)),
            pl.BlockSpec((3, H), lambda i: (0, 0)),
            pl.BlockSpec((1, H), lambda i: (0, 0)),
            pl.BlockSpec((H, H), lambda i: (0, 0)),
            pl.BlockSpec((1, H), lambda i: (0, 0)),
            pl.BlockSpec((320, OUT), lambda i: (0, 0)),
            pl.BlockSpec((1, OUT), lambda i: (0, 0)),
            pl.BlockSpec((OUT, OUT), lambda i: (0, 0)),
            pl.BlockSpec((1, OUT), lambda i: (0, 0)),
        ],
        out_specs=pl.BlockSpec((BK, OUT), lambda i: (i, 0)),
        out_shape=jax.ShapeDtypeStruct((B, OUT), f32),
    )(g3, target_obj_pos, receptacle_obj_pos,
      W_sub, b_sub.reshape(1, H), W_obj, b_obj.reshape(1, H),
      W_pos1, b_pos1.reshape(1, H), W_pos2, b_pos2.reshape(1, H),
      W_c1, b_c1.reshape(1, OUT), W_c2, b_c2.reshape(1, OUT))
    return out


# speedup vs baseline: 1.8079x; 1.0819x over previous
import jax
import jax.numpy as jnp
from jax import lax
from jax.experimental import pallas as pl
from jax.experimental.pallas import tpu as pltpu
from jax.experimental.pallas import tpu_sc as plsc

B = 16384
H = 64
OUT = 128
SUB_V = 6
OBJ_V = 1000
NC = 2
NS = 16
NW = NC * NS
B_PER = B // NW
QUART = B_PER // 4
GRP = 16
STRIDE = H + 1
BK = 4096


def _sc_gather_body(sid, tid, rid, tcat, out,
                    tcat_sh, tcat_v, sid_v, tid_v, rid_v,
                    st0, to0, ro0, st1, to1, ro1, sem, wsem):
    sub = lax.axis_index("s")
    wid = sub * NC + lax.axis_index("c")
    base = wid * B_PER

    @pl.when(sub == 0)
    def _stage_shared():
        pltpu.sync_copy(tcat, tcat_sh)

    stage = [pltpu.async_copy(sid.at[pl.ds(base, B_PER)], sid_v, sem),
             pltpu.async_copy(tid.at[pl.ds(base, B_PER)], tid_v, sem),
             pltpu.async_copy(rid.at[pl.ds(base, B_PER)], rid_v, sem)]
    for c in stage:
        c.wait()
    plsc.subcore_barrier()
    pltpu.sync_copy(tcat_sh, tcat_v)
    bufsets = ((st0, to0, ro0), (st1, to1, ro1))
    wcopies = []
    for q in range(B_PER // QUART):
        bufs = bufsets[q % 2]
        if q >= 2:
            for _ in range(3):
                wcopies.pop(0).wait()

        @plsc.parallel_loop(0, QUART // GRP, unroll=4)
        def _gather_groups(g, _q=q, _bufs=bufs):
            off = _q * QUART + g * GRP
            for ids_ref, row_off, buf_ref in ((sid_v, 0, _bufs[0]),
                                              (tid_v, SUB_V, _bufs[1]),
                                              (rid_v, SUB_V, _bufs[2])):
                fidx = (ids_ref[pl.ds(off, GRP)] + row_off) * STRIDE
                for c0 in range(0, H, 16):
                    xs = [plsc.load_gather(tcat_v, [fidx + c])
                          for c in range(c0, c0 + 16)]
                    for k, x in enumerate(xs):
                        buf_ref[c0 + k, pl.ds(g * GRP, GRP)] = x
        dst = pl.ds(base + q * QUART, QUART)
        for t in range(3):
            wcopies.append(pltpu.async_copy(bufs[t], out.at[t, :, dst], wsem))
    for c in wcopies:
        c.wait()


def _dot_t(a, w):
    return lax.dot_general(a, w, (((0,), (0,)), ((), ())),
                           preferred_element_type=jnp.float32)


def _dot(a, w):
    return jnp.dot(a, w, preferred_element_type=jnp.float32)


def _tail_body(g3, tpos, rpos, w_sub, b_sub, w_obj, b_obj,
               w_pos1, b_pos1, w_pos2, b_pos2, w_c1, b_c1,
               w_c2, b_c2, out):
    wc1_s = w_c1[0:64, :]
    wc1_t = w_c1[64:128, :]
    wc1_p1 = w_c1[128:192, :]
    wc1_r = w_c1[192:256, :]
    wc1_p2 = w_c1[256:320, :]
    a_s = _dot(w_sub[...], wc1_s)
    a_t = _dot(w_obj[...], wc1_t)
    a_r = _dot(w_obj[...], wc1_r)
    m_t = _dot(w_pos2[...], wc1_p1)
    m_r = _dot(w_pos2[...], wc1_p2)
    cvec = (b_c1[...]
            + _dot(b_sub[...], wc1_s)
            + _dot(b_obj[...], wc1_t) + _dot(b_obj[...], wc1_r)
            + _dot(b_pos2[...], wc1_p1) + _dot(b_pos2[...], wc1_p2))
    at_ = jnp.maximum(_dot(tpos[...], w_pos1[...]) + b_pos1[...], 0.0)
    ar_ = jnp.maximum(_dot(rpos[...], w_pos1[...]) + b_pos1[...], 0.0)
    bf16 = jnp.bfloat16
    h1 = (_dot_t(jnp.maximum(g3[0], 0.0).astype(bf16), a_s.astype(bf16))
          + _dot_t(jnp.maximum(g3[1], 0.0).astype(bf16), a_t.astype(bf16))
          + _dot_t(jnp.maximum(g3[2], 0.0).astype(bf16), a_r.astype(bf16))
          + _dot(at_, m_t)
          + _dot(ar_, m_r)
          + cvec)
    h = jnp.maximum(h1, 0.0)
    out[...] = jnp.maximum(
        _dot(h.astype(bf16), w_c2[...].astype(bf16)) + b_c2[...], 0.0)


def kernel(subtask_type_id, target_obj_type_id, target_obj_pos,
           receptacle_obj_type_id, receptacle_obj_pos,
           emb_sub, W_sub, b_sub, emb_obj, W_obj, b_obj,
           W_pos1, b_pos1, W_pos2, b_pos2, W_c1, b_c1, W_c2, b_c2):
    f32 = jnp.float32

    ecat = jnp.pad(jnp.concatenate([emb_sub, emb_obj], axis=0),
                   ((0, 0), (0, 1))).reshape(-1)

    mesh = plsc.VectorSubcoreMesh(core_axis_name="c", subcore_axis_name="s")
    g3 = pl.kernel(
        _sc_gather_body,
        out_type=jax.ShapeDtypeStruct((3, H, B), f32),
        mesh=mesh,
        compiler_params=pltpu.CompilerParams(needs_layout_passes=False),
        scratch_types=[
            pltpu.VMEM_SHARED(((SUB_V + OBJ_V) * STRIDE,), f32),
            pltpu.VMEM(((SUB_V + OBJ_V) * STRIDE,), f32),
            pltpu.VMEM((B_PER,), jnp.int32),
            pltpu.VMEM((B_PER,), jnp.int32),
            pltpu.VMEM((B_PER,), jnp.int32),
            pltpu.VMEM((H, QUART), f32),
            pltpu.VMEM((H, QUART), f32),
            pltpu.VMEM((H, QUART), f32),
            pltpu.VMEM((H, QUART), f32),
            pltpu.VMEM((H, QUART), f32),
            pltpu.VMEM((H, QUART), f32),
            pltpu.SemaphoreType.DMA,
            pltpu.SemaphoreType.DMA,
        ],
    )(subtask_type_id, target_obj_type_id, receptacle_obj_type_id, ecat)

    grid = (B // BK,)
    out = pl.pallas_call(
        _tail_body,
        grid=grid,
        in_specs=[
            pl.BlockSpec((3, H, BK), lambda i: (0, 0, i)),
            pl.BlockSpec((BK, 3), lambda i: (i, 0)),
            pl.BlockSpec((BK, 3), lambda i: (i, 0)),
            pl.BlockSpec((H, H), lambda i: (0, 0)),
            pl.BlockSpec((1, H), lambda i: (0, 0)),
            pl.BlockSpec((H, H), lambda i: (0, 0)),
            pl.BlockSpec((1, H), lambda i: (0, 0)),
            pl.BlockSpec((3, H), lambda i: (0, 0)),
            pl.BlockSpec((1, H), lambda i: (0, 0)),
            pl.BlockSpec((H, H), lambda i: (0, 0)),
            pl.BlockSpec((1, H), lambda i: (0, 0)),
            pl.BlockSpec((320, OUT), lambda i: (0, 0)),
            pl.BlockSpec((1, OUT), lambda i: (0, 0)),
            pl.BlockSpec((OUT, OUT), lambda i: (0, 0)),
            pl.BlockSpec((1, OUT), lambda i: (0, 0)),
        ],
        out_specs=pl.BlockSpec((BK, OUT), lambda i: (i, 0)),
        out_shape=jax.ShapeDtypeStruct((B, OUT), f32),
    )(g3, target_obj_pos, receptacle_obj_pos,
      W_sub, b_sub.reshape(1, H), W_obj, b_obj.reshape(1, H),
      W_pos1, b_pos1.reshape(1, H), W_pos2, b_pos2.reshape(1, H),
      W_c1, b_c1.reshape(1, OUT), W_c2, b_c2.reshape(1, OUT))
    return out
